# baseline (speedup 1.0000x reference)
import functools

import jax
import jax.numpy as jnp
from jax import lax
from jax.experimental import pallas as pl
from jax.experimental.pallas import tpu as pltpu

T, H, I, E, K = 2048, 2048, 2048, 8, 2
EPS = 1e-05
ALPHA = 1.702
LIMIT = 7.0

BM = 128
NB = (T * K) // BM + E
P = NB * BM
TA = 256
IBN = 1024


def _norm_gate_body(x_ref, scale_ref, gw_ref, gb_ref, t_ref, idx_ref, prob_ref):
    xb = x_ref[...]
    ms = jnp.mean(xb * xb, axis=1, keepdims=True)
    tb = xb * lax.rsqrt(ms + EPS) * scale_ref[...]
    t_ref[...] = tb.astype(jnp.bfloat16)
    g = lax.dot_general(tb, gw_ref[...], (((1,), (1,)), ((), ())),
                        preferred_element_type=jnp.float32) + gb_ref[...]
    iota = lax.broadcasted_iota(jnp.int32, g.shape, 1)
    v1 = jnp.max(g, axis=1, keepdims=True)
    i1 = jnp.min(jnp.where(g == v1, iota, E), axis=1, keepdims=True)
    g2 = jnp.where(iota == i1, -jnp.inf, g)
    v2 = jnp.max(g2, axis=1, keepdims=True)
    i2 = jnp.min(jnp.where(g2 == v2, iota, E), axis=1, keepdims=True)
    s = jnp.exp(v2 - v1)
    p1 = 1.0 / (1.0 + s)
    idx_ref[...] = jnp.concatenate([i1, i2], axis=1)
    prob_ref[...] = jnp.concatenate([p1, 1.0 - p1], axis=1)


def _norm_gate(x, norm_scale, gate_w, gate_b):
    return pl.pallas_call(
        _norm_gate_body,
        grid=(T // TA,),
        in_specs=[
            pl.BlockSpec((TA, H), lambda i: (i, 0)),
            pl.BlockSpec((1, H), lambda i: (0, 0)),
            pl.BlockSpec((E, H), lambda i: (0, 0)),
            pl.BlockSpec((1, E), lambda i: (0, 0)),
        ],
        out_specs=[
            pl.BlockSpec((TA, H), lambda i: (i, 0)),
            pl.BlockSpec((TA, K), lambda i: (i, 0)),
            pl.BlockSpec((TA, K), lambda i: (i, 0)),
        ],
        out_shape=[
            jax.ShapeDtypeStruct((T, H), jnp.bfloat16),
            jax.ShapeDtypeStruct((T, K), jnp.int32),
            jax.ShapeDtypeStruct((T, K), jnp.float32),
        ],
    )(x, norm_scale.reshape(1, H), gate_w, gate_b.reshape(1, E))


def _routing(idx, probs):
    e_pairs = idx.reshape(-1)
    tok = (jnp.arange(T * K, dtype=jnp.int32) // K).astype(jnp.int32)
    onehot = (e_pairs[:, None] == jnp.arange(E, dtype=e_pairs.dtype)[None, :]
              ).astype(jnp.int32)
    csum = jnp.cumsum(onehot, axis=0)
    counts = csum[-1]
    rank = jnp.take_along_axis(csum, e_pairs[:, None], axis=1)[:, 0] - 1
    padded = ((counts + BM - 1) // BM) * BM
    seg_start = jnp.concatenate(
        [jnp.zeros((1,), jnp.int32), jnp.cumsum(padded)])[:E]
    pos = seg_start[e_pairs] + rank
    perm = jnp.zeros((P,), jnp.int32).at[pos].set(tok)
    psort = jnp.zeros((P,), jnp.float32).at[pos].set(probs.reshape(-1))
    seg_end = seg_start + padded
    block_start = jnp.arange(NB, dtype=jnp.int32) * BM
    block_expert = jnp.minimum(
        jnp.sum((block_start[:, None] >= seg_end[None, :]).astype(jnp.int32),
                axis=1), E - 1).astype(jnp.int32)
    return perm, psort, pos.reshape(T, K), block_expert


def _gmm1_body(eb_ref, t_ref, wg_ref, wl_ref, bg_ref, bl_ref, h_ref):
    tb = t_ref[...]
    u = lax.dot_general(tb, wg_ref[0], (((1,), (1,)), ((), ())),
                        preferred_element_type=jnp.float32) + bg_ref[0]
    v = lax.dot_general(tb, wl_ref[0], (((1,), (1,)), ((), ())),
                        preferred_element_type=jnp.float32) + bl_ref[0]
    xg = jnp.minimum(u, LIMIT)
    xl = jnp.clip(v, -LIMIT, LIMIT)
    h = (xg * jax.nn.sigmoid(ALPHA * xg)) * (xl + 1.0)
    h_ref[...] = h.astype(jnp.bfloat16)


def _gmm1(t_sorted, W1g, W1l, b1g, b1l, block_expert):
    return pl.pallas_call(
        _gmm1_body,
        grid_spec=pltpu.PrefetchScalarGridSpec(
            num_scalar_prefetch=1,
            grid=(I // IBN, NB),
            in_specs=[
                pl.BlockSpec((BM, H), lambda n, m, eb: (m, 0)),
                pl.BlockSpec((1, IBN, H), lambda n, m, eb: (eb[m], n, 0)),
                pl.BlockSpec((1, IBN, H), lambda n, m, eb: (eb[m], n, 0)),
                pl.BlockSpec((1, 1, IBN), lambda n, m, eb: (eb[m], 0, n)),
                pl.BlockSpec((1, 1, IBN), lambda n, m, eb: (eb[m], 0, n)),
            ],
            out_specs=pl.BlockSpec((BM, IBN), lambda n, m, eb: (m, n)),
        ),
        out_shape=jax.ShapeDtypeStruct((P, I), jnp.bfloat16),
        compiler_params=pltpu.CompilerParams(
            dimension_semantics=("arbitrary", "arbitrary")),
    )(block_expert, t_sorted, W1g, W1l, b1g, b1l)


def _gmm2_body(eb_ref, h_ref, w2_ref, b2_ref, p_ref, y_ref):
    acc = lax.dot_general(h_ref[...], w2_ref[0], (((1,), (1,)), ((), ())),
                          preferred_element_type=jnp.float32) + b2_ref[0]
    y_ref[...] = acc * p_ref[...]


def _gmm2(h_sorted, W2, b2, psort, block_expert):
    return pl.pallas_call(
        _gmm2_body,
        grid_spec=pltpu.PrefetchScalarGridSpec(
            num_scalar_prefetch=1,
            grid=(NB,),
            in_specs=[
                pl.BlockSpec((BM, I), lambda m, eb: (m, 0)),
                pl.BlockSpec((1, H, I), lambda m, eb: (eb[m], 0, 0)),
                pl.BlockSpec((1, 1, H), lambda m, eb: (eb[m], 0, 0)),
                pl.BlockSpec((BM, 1), lambda m, eb: (m, 0)),
            ],
            out_specs=pl.BlockSpec((BM, H), lambda m, eb: (m, 0)),
        ),
        out_shape=jax.ShapeDtypeStruct((P, H), jnp.float32),
        compiler_params=pltpu.CompilerParams(
            dimension_semantics=("arbitrary",)),
    )(block_expert, h_sorted, W2, b2.reshape(E, 1, H), psort.reshape(P, 1))


def kernel(x, norm_scale, gate_w, gate_b, W1, b1, W2, b2):
    t, idx, probs = _norm_gate(x, norm_scale, gate_w, gate_b)
    perm, psort, pos, block_expert = _routing(idx, probs)
    t_sorted = t[perm]
    bf = jnp.bfloat16
    W1g, W1l = W1[:, 0::2, :].astype(bf), W1[:, 1::2, :].astype(bf)
    b1g = b1[:, 0::2].reshape(E, 1, I)
    b1l = b1[:, 1::2].reshape(E, 1, I)
    h_sorted = _gmm1(t_sorted, W1g, W1l, b1g, b1l, block_expert)
    y = _gmm2(h_sorted, W2.astype(bf), b2, psort, block_expert)
    out = x + y[pos[:, 0]] + y[pos[:, 1]]
    return out


# speedup vs baseline: 1.8834x; 1.8834x over previous
import functools

import jax
import jax.numpy as jnp
from jax import lax
from jax.experimental import pallas as pl
from jax.experimental.pallas import tpu as pltpu

T, H, I, E, K = 2048, 2048, 2048, 8, 2
EPS = 1e-05
ALPHA = 1.702
LIMIT = 7.0

BM = 128
NB = (T * K) // BM + E
P = NB * BM
TA = 256
IBN = 1024


def _norm_gate_body(x_ref, scale_ref, gw_ref, gb_ref, t_ref, idx_ref, prob_ref):
    xb = x_ref[...]
    ms = jnp.mean(xb * xb, axis=1, keepdims=True)
    tb = xb * lax.rsqrt(ms + EPS) * scale_ref[...]
    t_ref[...] = tb.astype(jnp.bfloat16)
    g = lax.dot_general(tb, gw_ref[...], (((1,), (1,)), ((), ())),
                        preferred_element_type=jnp.float32) + gb_ref[...]
    iota = lax.broadcasted_iota(jnp.int32, g.shape, 1)
    v1 = jnp.max(g, axis=1, keepdims=True)
    i1 = jnp.min(jnp.where(g == v1, iota, E), axis=1, keepdims=True)
    g2 = jnp.where(iota == i1, -jnp.inf, g)
    v2 = jnp.max(g2, axis=1, keepdims=True)
    i2 = jnp.min(jnp.where(g2 == v2, iota, E), axis=1, keepdims=True)
    s = jnp.exp(v2 - v1)
    p1 = 1.0 / (1.0 + s)
    idx_ref[...] = jnp.concatenate([i1, i2], axis=1)
    prob_ref[...] = jnp.concatenate([p1, 1.0 - p1], axis=1)


def _norm_gate(x, norm_scale, gate_w, gate_b):
    return pl.pallas_call(
        _norm_gate_body,
        grid=(T // TA,),
        in_specs=[
            pl.BlockSpec((TA, H), lambda i: (i, 0)),
            pl.BlockSpec((1, H), lambda i: (0, 0)),
            pl.BlockSpec((E, H), lambda i: (0, 0)),
            pl.BlockSpec((1, E), lambda i: (0, 0)),
        ],
        out_specs=[
            pl.BlockSpec((TA, H), lambda i: (i, 0)),
            pl.BlockSpec((TA, K), lambda i: (i, 0)),
            pl.BlockSpec((TA, K), lambda i: (i, 0)),
        ],
        out_shape=[
            jax.ShapeDtypeStruct((T, H), jnp.bfloat16),
            jax.ShapeDtypeStruct((T, K), jnp.int32),
            jax.ShapeDtypeStruct((T, K), jnp.float32),
        ],
    )(x, norm_scale.reshape(1, H), gate_w, gate_b.reshape(1, E))


def _routing(idx, probs):
    e_pairs = idx.reshape(-1)
    tok = (jnp.arange(T * K, dtype=jnp.int32) // K).astype(jnp.int32)
    onehot = (e_pairs[:, None] == jnp.arange(E, dtype=e_pairs.dtype)[None, :]
              ).astype(jnp.int32)
    csum = jnp.cumsum(onehot, axis=0)
    counts = csum[-1]
    rank = jnp.take_along_axis(csum, e_pairs[:, None], axis=1)[:, 0] - 1
    padded = ((counts + BM - 1) // BM) * BM
    seg_start = jnp.concatenate(
        [jnp.zeros((1,), jnp.int32), jnp.cumsum(padded)])[:E]
    pos = seg_start[e_pairs] + rank
    perm = jnp.zeros((P,), jnp.int32).at[pos].set(tok)
    psort = jnp.zeros((P,), jnp.float32).at[pos].set(probs.reshape(-1))
    seg_end = seg_start + padded
    block_start = jnp.arange(NB, dtype=jnp.int32) * BM
    block_expert = jnp.minimum(
        jnp.sum((block_start[:, None] >= seg_end[None, :]).astype(jnp.int32),
                axis=1), E - 1).astype(jnp.int32)
    return perm, psort, pos.reshape(T, K), block_expert


def _gmm1_body(eb_ref, t_ref, wg_ref, wl_ref, bg_ref, bl_ref, h_ref):
    tb = t_ref[...]
    u = lax.dot_general(tb, wg_ref[0], (((1,), (1,)), ((), ())),
                        preferred_element_type=jnp.float32) + bg_ref[0]
    v = lax.dot_general(tb, wl_ref[0], (((1,), (1,)), ((), ())),
                        preferred_element_type=jnp.float32) + bl_ref[0]
    xg = jnp.minimum(u, LIMIT)
    xl = jnp.clip(v, -LIMIT, LIMIT)
    h = (xg * jax.nn.sigmoid(ALPHA * xg)) * (xl + 1.0)
    h_ref[...] = h.astype(jnp.bfloat16)


def _gmm1(t_sorted, W1v, b1g, b1l, block_expert):
    return pl.pallas_call(
        _gmm1_body,
        grid_spec=pltpu.PrefetchScalarGridSpec(
            num_scalar_prefetch=1,
            grid=(I // IBN, NB),
            in_specs=[
                pl.BlockSpec((BM, H), lambda n, m, eb: (m, 0)),
                pl.BlockSpec((1, IBN, H), lambda n, m, eb: (eb[m], n, 0)),
                pl.BlockSpec((1, IBN, H), lambda n, m, eb: (eb[m], n, 1)),
                pl.BlockSpec((1, 1, IBN), lambda n, m, eb: (eb[m], 0, n)),
                pl.BlockSpec((1, 1, IBN), lambda n, m, eb: (eb[m], 0, n)),
            ],
            out_specs=pl.BlockSpec((BM, IBN), lambda n, m, eb: (m, n)),
        ),
        out_shape=jax.ShapeDtypeStruct((P, I), jnp.bfloat16),
        compiler_params=pltpu.CompilerParams(
            dimension_semantics=("arbitrary", "arbitrary")),
    )(block_expert, t_sorted, W1v, W1v, b1g, b1l)


def _gmm2_body(eb_ref, h_ref, w2_ref, b2_ref, p_ref, y_ref):
    acc = lax.dot_general(h_ref[...], w2_ref[0], (((1,), (1,)), ((), ())),
                          preferred_element_type=jnp.float32) + b2_ref[0]
    y_ref[...] = acc * p_ref[...]


def _gmm2(h_sorted, W2, b2, psort, block_expert):
    return pl.pallas_call(
        _gmm2_body,
        grid_spec=pltpu.PrefetchScalarGridSpec(
            num_scalar_prefetch=1,
            grid=(NB,),
            in_specs=[
                pl.BlockSpec((BM, I), lambda m, eb: (m, 0)),
                pl.BlockSpec((1, H, I), lambda m, eb: (eb[m], 0, 0)),
                pl.BlockSpec((1, 1, H), lambda m, eb: (eb[m], 0, 0)),
                pl.BlockSpec((BM, 1), lambda m, eb: (m, 0)),
            ],
            out_specs=pl.BlockSpec((BM, H), lambda m, eb: (m, 0)),
        ),
        out_shape=jax.ShapeDtypeStruct((P, H), jnp.float32),
        compiler_params=pltpu.CompilerParams(
            dimension_semantics=("arbitrary",)),
    )(block_expert, h_sorted, W2, b2.reshape(E, 1, H), psort.reshape(P, 1))


def kernel(x, norm_scale, gate_w, gate_b, W1, b1, W2, b2):
    t, idx, probs = _norm_gate(x, norm_scale, gate_w, gate_b)
    perm, psort, pos, block_expert = _routing(idx, probs)
    t_sorted = t[perm]
    bf = jnp.bfloat16
    W1v = W1.reshape(E, I, 2 * H).astype(bf)
    b1g = b1[:, 0::2].reshape(E, 1, I)
    b1l = b1[:, 1::2].reshape(E, 1, I)
    h_sorted = _gmm1(t_sorted, W1v, b1g, b1l, block_expert)
    y = _gmm2(h_sorted, W2.astype(bf), b2, psort, block_expert)
    out = x + y[pos[:, 0]] + y[pos[:, 1]]
    return out


# speedup vs baseline: 2.0046x; 1.0643x over previous
import functools

import jax
import jax.numpy as jnp
from jax import lax
from jax.experimental import pallas as pl
from jax.experimental.pallas import tpu as pltpu

T, H, I, E, K = 2048, 2048, 2048, 8, 2
EPS = 1e-05
ALPHA = 1.702
LIMIT = 7.0

BM = 128
NB = (T * K) // BM + E
P = NB * BM
TA = 256
IBN = 1024


def _norm_gate_body(x_ref, scale_ref, gw_ref, gb_ref, t_ref, idx_ref, prob_ref):
    xb = x_ref[...]
    ms = jnp.mean(xb * xb, axis=1, keepdims=True)
    tb = xb * lax.rsqrt(ms + EPS) * scale_ref[...]
    t_ref[...] = tb
    g = lax.dot_general(tb, gw_ref[...], (((1,), (1,)), ((), ())),
                        preferred_element_type=jnp.float32) + gb_ref[...]
    iota = lax.broadcasted_iota(jnp.int32, g.shape, 1)
    v1 = jnp.max(g, axis=1, keepdims=True)
    i1 = jnp.min(jnp.where(g == v1, iota, E), axis=1, keepdims=True)
    g2 = jnp.where(iota == i1, -jnp.inf, g)
    v2 = jnp.max(g2, axis=1, keepdims=True)
    i2 = jnp.min(jnp.where(g2 == v2, iota, E), axis=1, keepdims=True)
    s = jnp.exp(v2 - v1)
    p1 = 1.0 / (1.0 + s)
    idx_ref[...] = jnp.concatenate([i1, i2], axis=1)
    prob_ref[...] = jnp.concatenate([p1, 1.0 - p1], axis=1)


def _norm_gate(x, norm_scale, gate_w, gate_b):
    return pl.pallas_call(
        _norm_gate_body,
        grid=(T // TA,),
        in_specs=[
            pl.BlockSpec((TA, H), lambda i: (i, 0)),
            pl.BlockSpec((1, H), lambda i: (0, 0)),
            pl.BlockSpec((E, H), lambda i: (0, 0)),
            pl.BlockSpec((1, E), lambda i: (0, 0)),
        ],
        out_specs=[
            pl.BlockSpec((TA, H), lambda i: (i, 0)),
            pl.BlockSpec((TA, K), lambda i: (i, 0)),
            pl.BlockSpec((TA, K), lambda i: (i, 0)),
        ],
        out_shape=[
            jax.ShapeDtypeStruct((T, H), jnp.float32),
            jax.ShapeDtypeStruct((T, K), jnp.int32),
            jax.ShapeDtypeStruct((T, K), jnp.float32),
        ],
    )(x, norm_scale.reshape(1, H), gate_w, gate_b.reshape(1, E))


def _routing(idx, probs):
    e_pairs = idx.reshape(-1)
    tok = (jnp.arange(T * K, dtype=jnp.int32) // K).astype(jnp.int32)
    onehot = (e_pairs[:, None] == jnp.arange(E, dtype=e_pairs.dtype)[None, :]
              ).astype(jnp.int32)
    csum = jnp.cumsum(onehot, axis=0)
    counts = csum[-1]
    rank = jnp.take_along_axis(csum, e_pairs[:, None], axis=1)[:, 0] - 1
    padded = ((counts + BM - 1) // BM) * BM
    seg_start = jnp.concatenate(
        [jnp.zeros((1,), jnp.int32), jnp.cumsum(padded)])[:E]
    pos = seg_start[e_pairs] + rank
    perm = jnp.zeros((P,), jnp.int32).at[pos].set(tok)
    psort = jnp.zeros((P,), jnp.float32).at[pos].set(probs.reshape(-1))
    seg_end = seg_start + padded
    block_start = jnp.arange(NB, dtype=jnp.int32) * BM
    block_expert = jnp.minimum(
        jnp.sum((block_start[:, None] >= seg_end[None, :]).astype(jnp.int32),
                axis=1), E - 1).astype(jnp.int32)
    return perm, psort, pos.reshape(T, K), block_expert


def _gmm1_body(eb_ref, t_ref, wg_ref, wl_ref, bg_ref, bl_ref, h_ref):
    tb = t_ref[...]
    u = lax.dot_general(tb, wg_ref[0], (((1,), (1,)), ((), ())),
                        preferred_element_type=jnp.float32) + bg_ref[0]
    v = lax.dot_general(tb, wl_ref[0], (((1,), (1,)), ((), ())),
                        preferred_element_type=jnp.float32) + bl_ref[0]
    xg = jnp.minimum(u, LIMIT)
    xl = jnp.clip(v, -LIMIT, LIMIT)
    h = (xg * jax.nn.sigmoid(ALPHA * xg)) * (xl + 1.0)
    h_ref[...] = h


def _gmm1(t_sorted, W1v, b1g, b1l, block_expert):
    return pl.pallas_call(
        _gmm1_body,
        grid_spec=pltpu.PrefetchScalarGridSpec(
            num_scalar_prefetch=1,
            grid=(I // IBN, NB),
            in_specs=[
                pl.BlockSpec((BM, H), lambda n, m, eb: (m, 0)),
                pl.BlockSpec((1, IBN, H), lambda n, m, eb: (eb[m], n, 0)),
                pl.BlockSpec((1, IBN, H), lambda n, m, eb: (eb[m], n, 1)),
                pl.BlockSpec((1, 1, IBN), lambda n, m, eb: (eb[m], 0, n)),
                pl.BlockSpec((1, 1, IBN), lambda n, m, eb: (eb[m], 0, n)),
            ],
            out_specs=pl.BlockSpec((BM, IBN), lambda n, m, eb: (m, n)),
        ),
        out_shape=jax.ShapeDtypeStruct((P, I), jnp.float32),
        compiler_params=pltpu.CompilerParams(
            dimension_semantics=("arbitrary", "arbitrary")),
    )(block_expert, t_sorted, W1v, W1v, b1g, b1l)


def _gmm2_body(eb_ref, h_ref, w2_ref, b2_ref, p_ref, y_ref):
    acc = lax.dot_general(h_ref[...], w2_ref[0], (((1,), (1,)), ((), ())),
                          preferred_element_type=jnp.float32) + b2_ref[0]
    y_ref[...] = acc * p_ref[...]


def _gmm2(h_sorted, W2, b2, psort, block_expert):
    return pl.pallas_call(
        _gmm2_body,
        grid_spec=pltpu.PrefetchScalarGridSpec(
            num_scalar_prefetch=1,
            grid=(NB,),
            in_specs=[
                pl.BlockSpec((BM, I), lambda m, eb: (m, 0)),
                pl.BlockSpec((1, H, I), lambda m, eb: (eb[m], 0, 0)),
                pl.BlockSpec((1, 1, H), lambda m, eb: (eb[m], 0, 0)),
                pl.BlockSpec((BM, 1), lambda m, eb: (m, 0)),
            ],
            out_specs=pl.BlockSpec((BM, H), lambda m, eb: (m, 0)),
        ),
        out_shape=jax.ShapeDtypeStruct((P, H), jnp.float32),
        compiler_params=pltpu.CompilerParams(
            dimension_semantics=("arbitrary",)),
    )(block_expert, h_sorted, W2, b2.reshape(E, 1, H), psort.reshape(P, 1))


def kernel(x, norm_scale, gate_w, gate_b, W1, b1, W2, b2):
    t, idx, probs = _norm_gate(x, norm_scale, gate_w, gate_b)
    perm, psort, pos, block_expert = _routing(idx, probs)
    t_sorted = t[perm]
    bf = jnp.bfloat16
    W1v = W1.reshape(E, I, 2 * H)
    b1g = b1[:, 0::2].reshape(E, 1, I)
    b1l = b1[:, 1::2].reshape(E, 1, I)
    h_sorted = _gmm1(t_sorted, W1v, b1g, b1l, block_expert)
    y = _gmm2(h_sorted, W2, b2, psort, block_expert)
    out = x + y[pos[:, 0]] + y[pos[:, 1]]
    return out


# speedup vs baseline: 2.3792x; 1.1869x over previous
import functools

import jax
import jax.numpy as jnp
from jax import lax
from jax.experimental import pallas as pl
from jax.experimental.pallas import tpu as pltpu

T, H, I, E, K = 2048, 2048, 2048, 8, 2
EPS = 1e-05
ALPHA = 1.702
LIMIT = 7.0

BM = 256
NB = (T * K) // BM + E
P = NB * BM
TA = 256
IBN = 1024


def _norm_gate_body(x_ref, scale_ref, gw_ref, gb_ref, t_ref, idx_ref, prob_ref):
    xb = x_ref[...]
    ms = jnp.mean(xb * xb, axis=1, keepdims=True)
    tb = xb * lax.rsqrt(ms + EPS) * scale_ref[...]
    t_ref[...] = tb
    g = lax.dot_general(tb, gw_ref[...], (((1,), (1,)), ((), ())),
                        preferred_element_type=jnp.float32) + gb_ref[...]
    iota = lax.broadcasted_iota(jnp.int32, g.shape, 1)
    v1 = jnp.max(g, axis=1, keepdims=True)
    i1 = jnp.min(jnp.where(g == v1, iota, E), axis=1, keepdims=True)
    g2 = jnp.where(iota == i1, -jnp.inf, g)
    v2 = jnp.max(g2, axis=1, keepdims=True)
    i2 = jnp.min(jnp.where(g2 == v2, iota, E), axis=1, keepdims=True)
    s = jnp.exp(v2 - v1)
    p1 = 1.0 / (1.0 + s)
    idx_ref[...] = jnp.concatenate([i1, i2], axis=1)
    prob_ref[...] = jnp.concatenate([p1, 1.0 - p1], axis=1)


def _norm_gate(x, norm_scale, gate_w, gate_b):
    return pl.pallas_call(
        _norm_gate_body,
        grid=(T // TA,),
        in_specs=[
            pl.BlockSpec((TA, H), lambda i: (i, 0)),
            pl.BlockSpec((1, H), lambda i: (0, 0)),
            pl.BlockSpec((E, H), lambda i: (0, 0)),
            pl.BlockSpec((1, E), lambda i: (0, 0)),
        ],
        out_specs=[
            pl.BlockSpec((TA, H), lambda i: (i, 0)),
            pl.BlockSpec((TA, K), lambda i: (i, 0)),
            pl.BlockSpec((TA, K), lambda i: (i, 0)),
        ],
        out_shape=[
            jax.ShapeDtypeStruct((T, H), jnp.float32),
            jax.ShapeDtypeStruct((T, K), jnp.int32),
            jax.ShapeDtypeStruct((T, K), jnp.float32),
        ],
    )(x, norm_scale.reshape(1, H), gate_w, gate_b.reshape(1, E))


def _routing(idx, probs):
    e_pairs = idx.reshape(-1)
    tok = (jnp.arange(T * K, dtype=jnp.int32) // K).astype(jnp.int32)
    onehot = (e_pairs[:, None] == jnp.arange(E, dtype=e_pairs.dtype)[None, :]
              ).astype(jnp.int32)
    csum = jnp.cumsum(onehot, axis=0)
    counts = csum[-1]
    rank = jnp.take_along_axis(csum, e_pairs[:, None], axis=1)[:, 0] - 1
    padded = ((counts + BM - 1) // BM) * BM
    seg_start = jnp.concatenate(
        [jnp.zeros((1,), jnp.int32), jnp.cumsum(padded)])[:E]
    pos = seg_start[e_pairs] + rank
    perm = jnp.zeros((P,), jnp.int32).at[pos].set(tok)
    psort = jnp.zeros((P,), jnp.float32).at[pos].set(probs.reshape(-1))
    seg_end = seg_start + padded
    block_start = jnp.arange(NB, dtype=jnp.int32) * BM
    block_expert = jnp.minimum(
        jnp.sum((block_start[:, None] >= seg_end[None, :]).astype(jnp.int32),
                axis=1), E - 1).astype(jnp.int32)
    return perm, psort, pos.reshape(T, K), block_expert


def _gmm1_body(eb_ref, t_ref, wg_ref, wl_ref, bg_ref, bl_ref, h_ref):
    tb = t_ref[...]
    u = lax.dot_general(tb, wg_ref[0], (((1,), (1,)), ((), ())),
                        preferred_element_type=jnp.float32) + bg_ref[0]
    v = lax.dot_general(tb, wl_ref[0], (((1,), (1,)), ((), ())),
                        preferred_element_type=jnp.float32) + bl_ref[0]
    xg = jnp.minimum(u, LIMIT)
    xl = jnp.clip(v, -LIMIT, LIMIT)
    h = (xg * jax.nn.sigmoid(ALPHA * xg)) * (xl + 1.0)
    h_ref[...] = h


def _gmm1(t_sorted, W1v, b1g, b1l, block_expert):
    return pl.pallas_call(
        _gmm1_body,
        grid_spec=pltpu.PrefetchScalarGridSpec(
            num_scalar_prefetch=1,
            grid=(I // IBN, NB),
            in_specs=[
                pl.BlockSpec((BM, H), lambda n, m, eb: (m, 0)),
                pl.BlockSpec((1, IBN, H), lambda n, m, eb: (eb[m], n, 0)),
                pl.BlockSpec((1, IBN, H), lambda n, m, eb: (eb[m], n, 1)),
                pl.BlockSpec((1, 1, IBN), lambda n, m, eb: (eb[m], 0, n)),
                pl.BlockSpec((1, 1, IBN), lambda n, m, eb: (eb[m], 0, n)),
            ],
            out_specs=pl.BlockSpec((BM, IBN), lambda n, m, eb: (m, n)),
        ),
        out_shape=jax.ShapeDtypeStruct((P, I), jnp.float32),
        compiler_params=pltpu.CompilerParams(
            dimension_semantics=("arbitrary", "arbitrary")),
    )(block_expert, t_sorted, W1v, W1v, b1g, b1l)


def _gmm2_body(eb_ref, h_ref, w2_ref, b2_ref, p_ref, y_ref):
    acc = lax.dot_general(h_ref[...], w2_ref[0], (((1,), (1,)), ((), ())),
                          preferred_element_type=jnp.float32) + b2_ref[0]
    y_ref[...] = acc * p_ref[...]


def _gmm2(h_sorted, W2, b2, psort, block_expert):
    return pl.pallas_call(
        _gmm2_body,
        grid_spec=pltpu.PrefetchScalarGridSpec(
            num_scalar_prefetch=1,
            grid=(NB,),
            in_specs=[
                pl.BlockSpec((BM, I), lambda m, eb: (m, 0)),
                pl.BlockSpec((1, H, I), lambda m, eb: (eb[m], 0, 0)),
                pl.BlockSpec((1, 1, H), lambda m, eb: (eb[m], 0, 0)),
                pl.BlockSpec((BM, 1), lambda m, eb: (m, 0)),
            ],
            out_specs=pl.BlockSpec((BM, H), lambda m, eb: (m, 0)),
        ),
        out_shape=jax.ShapeDtypeStruct((P, H), jnp.float32),
        compiler_params=pltpu.CompilerParams(
            dimension_semantics=("arbitrary",)),
    )(block_expert, h_sorted, W2, b2.reshape(E, 1, H), psort.reshape(P, 1))


def kernel(x, norm_scale, gate_w, gate_b, W1, b1, W2, b2):
    t, idx, probs = _norm_gate(x, norm_scale, gate_w, gate_b)
    perm, psort, pos, block_expert = _routing(idx, probs)
    t_sorted = t[perm]
    bf = jnp.bfloat16
    W1v = W1.reshape(E, I, 2 * H)
    b1g = b1[:, 0::2].reshape(E, 1, I)
    b1l = b1[:, 1::2].reshape(E, 1, I)
    h_sorted = _gmm1(t_sorted, W1v, b1g, b1l, block_expert)
    y = _gmm2(h_sorted, W2, b2, psort, block_expert)
    out = x + y[pos[:, 0]] + y[pos[:, 1]]
    return out


# speedup vs baseline: 3.1109x; 1.3075x over previous
import functools

import jax
import jax.numpy as jnp
from jax import lax
from jax.experimental import pallas as pl
from jax.experimental.pallas import tpu as pltpu

T, H, I, E, K = 2048, 2048, 2048, 8, 2
EPS = 1e-05
ALPHA = 1.702
LIMIT = 7.0

BM = 256
NB = (T * K) // BM + E
P = NB * BM
TA = 256
IBN = 1024


def _norm_gate_body(x_ref, scale_ref, gw_ref, gb_ref, t_ref, idx_ref, prob_ref):
    xb = x_ref[...]
    ms = jnp.mean(xb * xb, axis=1, keepdims=True)
    tb = xb * lax.rsqrt(ms + EPS) * scale_ref[...]
    t_ref[...] = tb
    g = lax.dot_general(tb, gw_ref[...], (((1,), (1,)), ((), ())),
                        preferred_element_type=jnp.float32) + gb_ref[...]
    iota = lax.broadcasted_iota(jnp.int32, g.shape, 1)
    v1 = jnp.max(g, axis=1, keepdims=True)
    i1 = jnp.min(jnp.where(g == v1, iota, E), axis=1, keepdims=True)
    g2 = jnp.where(iota == i1, -jnp.inf, g)
    v2 = jnp.max(g2, axis=1, keepdims=True)
    i2 = jnp.min(jnp.where(g2 == v2, iota, E), axis=1, keepdims=True)
    s = jnp.exp(v2 - v1)
    p1 = 1.0 / (1.0 + s)
    idx_ref[...] = jnp.concatenate([i1, i2], axis=1)
    prob_ref[...] = jnp.concatenate([p1, 1.0 - p1], axis=1)


def _norm_gate(x, norm_scale, gate_w, gate_b):
    return pl.pallas_call(
        _norm_gate_body,
        grid=(T // TA,),
        in_specs=[
            pl.BlockSpec((TA, H), lambda i: (i, 0)),
            pl.BlockSpec((1, H), lambda i: (0, 0)),
            pl.BlockSpec((E, H), lambda i: (0, 0)),
            pl.BlockSpec((1, E), lambda i: (0, 0)),
        ],
        out_specs=[
            pl.BlockSpec((TA, H), lambda i: (i, 0)),
            pl.BlockSpec((TA, K), lambda i: (i, 0)),
            pl.BlockSpec((TA, K), lambda i: (i, 0)),
        ],
        out_shape=[
            jax.ShapeDtypeStruct((T, H), jnp.float32),
            jax.ShapeDtypeStruct((T, K), jnp.int32),
            jax.ShapeDtypeStruct((T, K), jnp.float32),
        ],
    )(x, norm_scale.reshape(1, H), gate_w, gate_b.reshape(1, E))


def _routing(idx, probs):
    e_pairs = idx.reshape(-1)
    tok = (jnp.arange(T * K, dtype=jnp.int32) // K).astype(jnp.int32)
    onehot = (e_pairs[:, None] == jnp.arange(E, dtype=e_pairs.dtype)[None, :]
              ).astype(jnp.int32)
    csum = jnp.cumsum(onehot, axis=0)
    counts = csum[-1]
    rank = jnp.take_along_axis(csum, e_pairs[:, None], axis=1)[:, 0] - 1
    padded = ((counts + BM - 1) // BM) * BM
    seg_start = jnp.concatenate(
        [jnp.zeros((1,), jnp.int32), jnp.cumsum(padded)])[:E]
    pos = seg_start[e_pairs] + rank
    perm = jnp.zeros((P,), jnp.int32).at[pos].set(tok)
    psort = jnp.zeros((P,), jnp.float32).at[pos].set(probs.reshape(-1))
    seg_end = seg_start + padded
    block_start = jnp.arange(NB, dtype=jnp.int32) * BM
    block_expert = jnp.minimum(
        jnp.sum((block_start[:, None] >= seg_end[None, :]).astype(jnp.int32),
                axis=1), E - 1).astype(jnp.int32)
    return perm, psort, pos.reshape(T, K), block_expert


def _gmm1_body(eb_ref, t_ref, w_ref, bg_ref, bl_ref, h_ref):
    x1t = lax.dot_general(w_ref[0], t_ref[...], (((1,), (1,)), ((), ())),
                          preferred_element_type=jnp.float32)
    x1r = x1t.reshape(IBN, 2, BM)
    u = x1r[:, 0, :] + bg_ref[0]
    v = x1r[:, 1, :] + bl_ref[0]
    xg = jnp.minimum(u, LIMIT)
    xl = jnp.clip(v, -LIMIT, LIMIT)
    h = (xg * jax.nn.sigmoid(ALPHA * xg)) * (xl + 1.0)
    h_ref[...] = h


def _gmm1(t_sorted, W1v, b1g, b1l, block_expert):
    return pl.pallas_call(
        _gmm1_body,
        grid_spec=pltpu.PrefetchScalarGridSpec(
            num_scalar_prefetch=1,
            grid=(I // IBN, NB),
            in_specs=[
                pl.BlockSpec((BM, H), lambda n, m, eb: (m, 0)),
                pl.BlockSpec((1, 2 * IBN, H), lambda n, m, eb: (eb[m], n, 0)),
                pl.BlockSpec((1, IBN, 1), lambda n, m, eb: (eb[m], n, 0)),
                pl.BlockSpec((1, IBN, 1), lambda n, m, eb: (eb[m], n, 0)),
            ],
            out_specs=pl.BlockSpec((IBN, BM), lambda n, m, eb: (n, m)),
        ),
        out_shape=jax.ShapeDtypeStruct((I, P), jnp.float32),
        compiler_params=pltpu.CompilerParams(
            dimension_semantics=("arbitrary", "arbitrary")),
    )(block_expert, t_sorted, W1v, b1g, b1l)


def _gmm2_body(eb_ref, h_ref, w2_ref, b2_ref, p_ref, y_ref):
    acc = lax.dot_general(h_ref[...], w2_ref[0], (((0,), (1,)), ((), ())),
                          preferred_element_type=jnp.float32) + b2_ref[0]
    y_ref[...] = acc * p_ref[...]


def _gmm2(h_sorted, W2, b2, psort, block_expert):
    return pl.pallas_call(
        _gmm2_body,
        grid_spec=pltpu.PrefetchScalarGridSpec(
            num_scalar_prefetch=1,
            grid=(NB,),
            in_specs=[
                pl.BlockSpec((I, BM), lambda m, eb: (0, m)),
                pl.BlockSpec((1, H, I), lambda m, eb: (eb[m], 0, 0)),
                pl.BlockSpec((1, 1, H), lambda m, eb: (eb[m], 0, 0)),
                pl.BlockSpec((BM, 1), lambda m, eb: (m, 0)),
            ],
            out_specs=pl.BlockSpec((BM, H), lambda m, eb: (m, 0)),
        ),
        out_shape=jax.ShapeDtypeStruct((P, H), jnp.float32),
        compiler_params=pltpu.CompilerParams(
            dimension_semantics=("arbitrary",)),
    )(block_expert, h_sorted, W2, b2.reshape(E, 1, H), psort.reshape(P, 1))


def kernel(x, norm_scale, gate_w, gate_b, W1, b1, W2, b2):
    t, idx, probs = _norm_gate(x, norm_scale, gate_w, gate_b)
    perm, psort, pos, block_expert = _routing(idx, probs)
    t_sorted = t[perm]
    bf = jnp.bfloat16
    W1v = W1
    b1g = b1[:, 0::2].reshape(E, I, 1)
    b1l = b1[:, 1::2].reshape(E, I, 1)
    h_sorted = _gmm1(t_sorted, W1v, b1g, b1l, block_expert)
    y = _gmm2(h_sorted, W2, b2, psort, block_expert)
    out = x + y[pos[:, 0]] + y[pos[:, 1]]
    return out
